# Initial kernel scaffold; baseline (speedup 1.0000x reference)
import jax
import jax.numpy as jnp
from jax.experimental import pallas as pl


def kernel(input, target, epoch):
    raise NotImplementedError("write your pallas kernel here")


# speedup vs baseline: 2.6397x; 2.6397x over previous
import functools

import jax
import jax.numpy as jnp
import numpy as np
from jax.experimental import pallas as pl
from jax.experimental.pallas import tpu as pltpu

_C = 1000
_B = 16384
_STEP_EPOCH = 16


def _class_consts():
    ncl = np.array([int(100 * 0.1 ** (i / (_C - 1.0))) for i in range(_C)],
                   dtype=np.float64)
    s = np.log(ncl * (50.0 / ncl.min()))
    s = s * (1.0 / s.min())
    m = s[::-1] * (0.5 / s[::-1].max())
    bs = np.clip(s[::-1] * 1.0, 1.0, 50.0)
    return (s.astype(np.float32)[None, :], m.astype(np.float32)[None, :],
            bs.astype(np.float32)[None, :])


_S_NP, _M_NP, _BS_NP = _class_consts()


def _loss_body(ep_ref, t_ref, x_ref, s_ref, m_ref, b_ref, o_ref):
    x = x_ref[...]
    t = t_ref[...]
    col = jax.lax.broadcasted_iota(jnp.int32, x.shape, 1)
    onehot = col == t
    z = x * s_ref[...] - jnp.where(onehot, m_ref[...], 0.0)
    ag = jnp.sum(jnp.where(onehot, b_ref[...], 0.0), axis=1, keepdims=True)
    a = jnp.where(ep_ref[0, 0] < _STEP_EPOCH, jnp.float32(1.0), ag)
    z = z * a
    zmax = jnp.max(z, axis=1, keepdims=True)
    se = jnp.sum(jnp.exp(z - zmax), axis=1, keepdims=True)
    zt = jnp.sum(jnp.where(onehot, z, 0.0), axis=1, keepdims=True)
    nll = zmax + jnp.log(se) - zt
    part = jnp.sum(nll, axis=0, keepdims=True) * jnp.float32(1.0 / _B)

    @pl.when(pl.program_id(0) == 0)
    def _init():
        o_ref[...] = jnp.zeros_like(o_ref)

    o_ref[...] += part


@functools.partial(jax.jit, static_argnames=("rows",))
def _kps_loss(x, t, ep, rows=512):
    grid = _B // rows
    out = pl.pallas_call(
        _loss_body,
        grid=(grid,),
        in_specs=[
            pl.BlockSpec(memory_space=pltpu.SMEM),
            pl.BlockSpec((rows, 1), lambda i: (i, 0)),
            pl.BlockSpec((rows, _C), lambda i: (i, 0)),
            pl.BlockSpec((1, _C), lambda i: (0, 0)),
            pl.BlockSpec((1, _C), lambda i: (0, 0)),
            pl.BlockSpec((1, _C), lambda i: (0, 0)),
        ],
        out_specs=pl.BlockSpec((1, 1), lambda i: (0, 0)),
        out_shape=jax.ShapeDtypeStruct((1, 1), jnp.float32),
    )(ep, t, x, jnp.asarray(_S_NP), jnp.asarray(_M_NP), jnp.asarray(_BS_NP))
    return out[0, 0]


def kernel(input, target, epoch):
    t2 = target.astype(jnp.int32).reshape(_B, 1)
    ep = jnp.asarray(epoch, jnp.int32).reshape(1, 1)
    return _kps_loss(input, t2, ep)
